# baseline (speedup 1.0000x reference)
import functools

import jax
import jax.numpy as jnp
from jax.experimental import pallas as pl

_BM0 = 200
_BM = 400
_BM_PROJ = 1000
_CF32 = 4352
_WPAD = 5760


def _ln_relu_proj(acc, b_ref, g_ref, beta_ref, w_ref):
    h = acc + b_ref[...]
    mu = jnp.mean(h, axis=-1, keepdims=True)
    var = jnp.mean((h - mu) ** 2, axis=-1, keepdims=True)
    hn = (h - mu) / jnp.sqrt(var + 1e-5) * g_ref[...] + beta_ref[...]
    hr = jnp.maximum(hn, 0.0)
    return jnp.dot(hr, w_ref[...],
                   preferred_element_type=jnp.float32).astype(jnp.bfloat16)


def _proj_body(x_ref, w_ref, o_ref):
    o_ref[...] = jnp.dot(x_ref[...], w_ref[...],
                         preferred_element_type=jnp.float32
                         ).astype(jnp.bfloat16)


def _l0_body(adj_ref, p_ref, b_ref, g_ref, beta_ref, w_ref,
             o_ref, adjb_ref, *, pad):
    ab = adj_ref[...].astype(jnp.bfloat16)
    adjb_ref[...] = jnp.pad(ab[:, _CF32:], ((0, 0), (0, pad)))
    acc = jnp.dot(ab, p_ref[...], preferred_element_type=jnp.float32)
    o_ref[...] = _ln_relu_proj(acc, b_ref, g_ref, beta_ref, w_ref)


def _mid_body(adjf_ref, adjb_ref, plo_ref, phi_ref,
              b_ref, g_ref, beta_ref, w_ref, o_ref):
    acc = (jnp.dot(adjf_ref[...], plo_ref[...],
                   preferred_element_type=jnp.float32)
           + jnp.dot(adjb_ref[...], phi_ref[...],
                     preferred_element_type=jnp.float32))
    o_ref[...] = _ln_relu_proj(acc, b_ref, g_ref, beta_ref, w_ref)


def _final_body(adjf_ref, adjb_ref, plo_ref, phi_ref, b_ref, o_ref):
    acc = (jnp.dot(adjf_ref[...], plo_ref[...],
                   preferred_element_type=jnp.float32)
           + jnp.dot(adjb_ref[...], phi_ref[...],
                     preferred_element_type=jnp.float32))
    h = acc + b_ref[...]
    m = jnp.max(h, axis=-1, keepdims=True)
    e = jnp.exp(h - m)
    lse = jnp.log(jnp.sum(e, axis=-1, keepdims=True)) + m
    o_ref[...] = h - lse


def _hi_pad(p, n, pad):
    return jnp.pad(jax.lax.slice(p, (_CF32, 0), (n, p.shape[1])),
                   ((0, pad), (0, 0)))


def kernel(x, adj_t, W0, b0, W1, b1, W2, b2, g1, beta1, g2, beta2):
    n, d = x.shape
    pad = _WPAD - (n - _CF32)
    r = lambda v: v.reshape(1, -1)
    vec = lambda: pl.BlockSpec((1, d), lambda i: (0, 0))

    p0 = pl.pallas_call(
        _proj_body,
        grid=(n // _BM_PROJ,),
        in_specs=[
            pl.BlockSpec((_BM_PROJ, d), lambda i: (i, 0)),
            pl.BlockSpec((d, d), lambda i: (0, 0)),
        ],
        out_specs=pl.BlockSpec((_BM_PROJ, d), lambda i: (i, 0)),
        out_shape=jax.ShapeDtypeStruct((n, d), jnp.bfloat16),
    )(x, W0)

    p1, adj_b = pl.pallas_call(
        functools.partial(_l0_body, pad=pad),
        grid=(n // _BM0,),
        in_specs=[
            pl.BlockSpec((_BM0, n), lambda i: (i, 0)),
            pl.BlockSpec((n, d), lambda i: (0, 0)),
            vec(), vec(), vec(),
            pl.BlockSpec((d, d), lambda i: (0, 0)),
        ],
        out_specs=[
            pl.BlockSpec((_BM0, d), lambda i: (i, 0)),
            pl.BlockSpec((_BM0, _WPAD), lambda i: (i, 0)),
        ],
        out_shape=[
            jax.ShapeDtypeStruct((n, d), jnp.bfloat16),
            jax.ShapeDtypeStruct((n, _WPAD), jnp.bfloat16),
        ],
    )(adj_t, p0, r(b0), r(g1), r(beta1), W1)

    mixed_specs = [
        pl.BlockSpec((_BM, _CF32), lambda i: (i, 0)),
        pl.BlockSpec((_BM, _WPAD), lambda i: (i, 0)),
        pl.BlockSpec((_CF32, d), lambda i: (0, 0)),
        pl.BlockSpec((_WPAD, d), lambda i: (0, 0)),
    ]

    p2 = pl.pallas_call(
        _mid_body,
        grid=(n // _BM,),
        in_specs=mixed_specs + [vec(), vec(), vec(),
                                pl.BlockSpec((d, d), lambda i: (0, 0))],
        out_specs=pl.BlockSpec((_BM, d), lambda i: (i, 0)),
        out_shape=jax.ShapeDtypeStruct((n, d), jnp.bfloat16),
    )(adj_t, adj_b, p1, _hi_pad(p1, n, pad), r(b1), r(g2), r(beta2), W2)

    return pl.pallas_call(
        _final_body,
        grid=(n // _BM,),
        in_specs=mixed_specs + [vec()],
        out_specs=pl.BlockSpec((_BM, d), lambda i: (i, 0)),
        out_shape=jax.ShapeDtypeStruct((n, d), jnp.float32),
    )(adj_t, adj_b, p2, _hi_pad(p2, n, pad), r(b2))


# speedup vs baseline: 1.0263x; 1.0263x over previous
import functools

import jax
import jax.numpy as jnp
from jax.experimental import pallas as pl

_BM0 = 200
_BM = 400
_BM_PROJ = 1000
_CF32 = 4352


def _ln_relu_proj(acc, b_ref, g_ref, beta_ref, w_ref):
    h = acc + b_ref[...]
    mu = jnp.mean(h, axis=-1, keepdims=True)
    var = jnp.mean((h - mu) ** 2, axis=-1, keepdims=True)
    hn = (h - mu) / jnp.sqrt(var + 1e-5) * g_ref[...] + beta_ref[...]
    hr = jnp.maximum(hn, 0.0)
    return jnp.dot(hr, w_ref[...],
                   preferred_element_type=jnp.float32).astype(jnp.bfloat16)


def _proj_body(x_ref, w_ref, o_ref):
    o_ref[...] = jnp.dot(x_ref[...], w_ref[...],
                         preferred_element_type=jnp.float32
                         ).astype(jnp.bfloat16)


def _l0_body(adj_ref, p_ref, b_ref, g_ref, beta_ref, w_ref,
             o_ref, adjb_ref):
    ab = adj_ref[...].astype(jnp.bfloat16)
    adjb_ref[...] = ab[:, _CF32:]
    acc = jnp.dot(ab, p_ref[...], preferred_element_type=jnp.float32)
    o_ref[...] = _ln_relu_proj(acc, b_ref, g_ref, beta_ref, w_ref)


def _mid_body(adjf_ref, adjb_ref, p_ref, b_ref, g_ref, beta_ref, w_ref,
              o_ref):
    nhi = adjb_ref.shape[1]
    acc = (jnp.dot(adjf_ref[...], p_ref[:_CF32, :],
                   preferred_element_type=jnp.float32)
           + jnp.dot(adjb_ref[...], p_ref[pl.ds(_CF32, nhi), :],
                     preferred_element_type=jnp.float32))
    o_ref[...] = _ln_relu_proj(acc, b_ref, g_ref, beta_ref, w_ref)


def _final_body(adjf_ref, adjb_ref, p_ref, b_ref, o_ref):
    nhi = adjb_ref.shape[1]
    acc = (jnp.dot(adjf_ref[...], p_ref[:_CF32, :],
                   preferred_element_type=jnp.float32)
           + jnp.dot(adjb_ref[...], p_ref[pl.ds(_CF32, nhi), :],
                     preferred_element_type=jnp.float32))
    h = acc + b_ref[...]
    m = jnp.max(h, axis=-1, keepdims=True)
    e = jnp.exp(h - m)
    lse = jnp.log(jnp.sum(e, axis=-1, keepdims=True)) + m
    o_ref[...] = h - lse


def kernel(x, adj_t, W0, b0, W1, b1, W2, b2, g1, beta1, g2, beta2):
    n, d = x.shape
    whi = n - _CF32
    r = lambda v: v.reshape(1, -1)
    vec = lambda: pl.BlockSpec((1, d), lambda i: (0, 0))

    p0 = pl.pallas_call(
        _proj_body,
        grid=(n // _BM_PROJ,),
        in_specs=[
            pl.BlockSpec((_BM_PROJ, d), lambda i: (i, 0)),
            pl.BlockSpec((d, d), lambda i: (0, 0)),
        ],
        out_specs=pl.BlockSpec((_BM_PROJ, d), lambda i: (i, 0)),
        out_shape=jax.ShapeDtypeStruct((n, d), jnp.bfloat16),
    )(x, W0)

    p1, adj_b = pl.pallas_call(
        _l0_body,
        grid=(n // _BM0,),
        in_specs=[
            pl.BlockSpec((_BM0, n), lambda i: (i, 0)),
            pl.BlockSpec((n, d), lambda i: (0, 0)),
            vec(), vec(), vec(),
            pl.BlockSpec((d, d), lambda i: (0, 0)),
        ],
        out_specs=[
            pl.BlockSpec((_BM0, d), lambda i: (i, 0)),
            pl.BlockSpec((_BM0, whi), lambda i: (i, 0)),
        ],
        out_shape=[
            jax.ShapeDtypeStruct((n, d), jnp.bfloat16),
            jax.ShapeDtypeStruct((n, whi), jnp.bfloat16),
        ],
    )(adj_t, p0, r(b0), r(g1), r(beta1), W1)

    mixed_specs = [
        pl.BlockSpec((_BM, _CF32), lambda i: (i, 0)),
        pl.BlockSpec((_BM, whi), lambda i: (i, 0)),
        pl.BlockSpec((n, d), lambda i: (0, 0)),
    ]

    p2 = pl.pallas_call(
        _mid_body,
        grid=(n // _BM,),
        in_specs=mixed_specs + [vec(), vec(), vec(),
                                pl.BlockSpec((d, d), lambda i: (0, 0))],
        out_specs=pl.BlockSpec((_BM, d), lambda i: (i, 0)),
        out_shape=jax.ShapeDtypeStruct((n, d), jnp.bfloat16),
    )(adj_t, adj_b, p1, r(b1), r(g2), r(beta2), W2)

    return pl.pallas_call(
        _final_body,
        grid=(n // _BM,),
        in_specs=mixed_specs + [vec()],
        out_specs=pl.BlockSpec((_BM, d), lambda i: (i, 0)),
        out_shape=jax.ShapeDtypeStruct((n, d), jnp.float32),
    )(adj_t, adj_b, p2, r(b2))


# speedup vs baseline: 1.0274x; 1.0011x over previous
import jax
import jax.numpy as jnp
from jax.experimental import pallas as pl

_BM0 = 200
_BM = 400
_BM_PROJ = 1000
_CF32 = 4352


def _ln_relu_proj(acc, b_ref, g_ref, beta_ref, w_ref):
    h = acc + b_ref[...]
    mu = jnp.mean(h, axis=-1, keepdims=True)
    var = jnp.mean((h - mu) ** 2, axis=-1, keepdims=True)
    hn = (h - mu) / jnp.sqrt(var + 1e-5) * g_ref[...] + beta_ref[...]
    hr = jnp.maximum(hn, 0.0)
    return jnp.dot(hr, w_ref[...],
                   preferred_element_type=jnp.float32).astype(jnp.bfloat16)


def _proj_body(x_ref, w_ref, o_ref):
    o_ref[...] = jnp.dot(x_ref[...], w_ref[...],
                         preferred_element_type=jnp.float32
                         ).astype(jnp.bfloat16)


def _l0_body(adj_ref, p_ref, b_ref, g_ref, beta_ref, w_ref,
             o_ref, adjb_ref):
    ab = adj_ref[...].astype(jnp.bfloat16)
    adjb_ref[...] = ab[:, _CF32:]
    acc = jnp.dot(ab, p_ref[...], preferred_element_type=jnp.float32)
    o_ref[...] = _ln_relu_proj(acc, b_ref, g_ref, beta_ref, w_ref)


def _mid_body(adjf_ref, adjb_ref, p_ref, b_ref, g_ref, beta_ref, w_ref,
              o_ref):
    nhi = adjb_ref.shape[1]
    acc = (jnp.dot(adjf_ref[...], p_ref[:_CF32, :],
                   preferred_element_type=jnp.float32)
           + jnp.dot(adjb_ref[...], p_ref[pl.ds(_CF32, nhi), :],
                     preferred_element_type=jnp.float32))
    o_ref[...] = _ln_relu_proj(acc, b_ref, g_ref, beta_ref, w_ref)


def _final_body(adjf_ref, adjb_ref, p_ref, b_ref, o_ref):
    nhi = adjb_ref.shape[1]
    acc = (jnp.dot(adjf_ref[...], p_ref[:_CF32, :],
                   preferred_element_type=jnp.float32)
           + jnp.dot(adjb_ref[...], p_ref[pl.ds(_CF32, nhi), :],
                     preferred_element_type=jnp.float32))
    h = acc + b_ref[...]
    m = jnp.max(h, axis=-1, keepdims=True)
    e = jnp.exp(h - m)
    lse = jnp.log(jnp.sum(e, axis=-1, keepdims=True)) + m
    o_ref[...] = h - lse


def kernel(x, adj_t, W0, b0, W1, b1, W2, b2, g1, beta1, g2, beta2):
    n, d = x.shape
    whi = n - _CF32
    r = lambda v: v.reshape(1, -1)
    vec = lambda: pl.BlockSpec((1, d), lambda i: (0, 0))

    p0 = pl.pallas_call(
        _proj_body,
        grid=(n // _BM_PROJ,),
        in_specs=[
            pl.BlockSpec((_BM_PROJ, d), lambda i: (i, 0)),
            pl.BlockSpec((d, d), lambda i: (0, 0)),
        ],
        out_specs=pl.BlockSpec((_BM_PROJ, d), lambda i: (i, 0)),
        out_shape=jax.ShapeDtypeStruct((n, d), jnp.bfloat16),
    )(x, W0)

    p1, adj_b = pl.pallas_call(
        _l0_body,
        grid=(n // _BM0,),
        in_specs=[
            pl.BlockSpec((_BM0, n), lambda i: (i, 0)),
            pl.BlockSpec((n, d), lambda i: (0, 0)),
            vec(), vec(), vec(),
            pl.BlockSpec((d, d), lambda i: (0, 0)),
        ],
        out_specs=[
            pl.BlockSpec((_BM0, d), lambda i: (i, 0)),
            pl.BlockSpec((_BM0, whi), lambda i: (i, 0)),
        ],
        out_shape=[
            jax.ShapeDtypeStruct((n, d), jnp.bfloat16),
            jax.ShapeDtypeStruct((n, whi), jnp.bfloat16),
        ],
    )(adj_t, p0, r(b0), r(g1), r(beta1), W1)

    mixed_specs = [
        pl.BlockSpec((_BM, _CF32), lambda i: (i, 0)),
        pl.BlockSpec((_BM, whi), lambda i: (i, 0)),
        pl.BlockSpec((n, d), lambda i: (0, 0)),
    ]

    p2 = pl.pallas_call(
        _mid_body,
        grid=(n // _BM,),
        in_specs=mixed_specs + [vec(), vec(), vec(),
                                pl.BlockSpec((d, d), lambda i: (0, 0))],
        out_specs=pl.BlockSpec((_BM, d), lambda i: (i, 0)),
        out_shape=jax.ShapeDtypeStruct((n, d), jnp.bfloat16),
    )(adj_t, adj_b, p1, r(b1), r(g2), r(beta2), W2)

    return pl.pallas_call(
        _final_body,
        grid=(n // _BM,),
        in_specs=mixed_specs + [vec()],
        out_specs=pl.BlockSpec((_BM, d), lambda i: (i, 0)),
        out_shape=jax.ShapeDtypeStruct((n, d), jnp.float32),
    )(adj_t, adj_b, p2, r(b2))


# speedup vs baseline: 1.0523x; 1.0242x over previous
import jax
import jax.numpy as jnp
from jax.experimental import pallas as pl

_BM0 = 400
_BM = 400
_BM_PROJ = 1000
_CF32 = 4352


def _ln_relu_proj(acc, b_ref, g_ref, beta_ref, w_ref):
    h = acc + b_ref[...]
    mu = jnp.mean(h, axis=-1, keepdims=True)
    var = jnp.mean((h - mu) ** 2, axis=-1, keepdims=True)
    hn = (h - mu) / jnp.sqrt(var + 1e-5) * g_ref[...] + beta_ref[...]
    hr = jnp.maximum(hn, 0.0)
    return jnp.dot(hr, w_ref[...],
                   preferred_element_type=jnp.float32).astype(jnp.bfloat16)


def _proj_body(x_ref, w_ref, o_ref):
    o_ref[...] = jnp.dot(x_ref[...], w_ref[...],
                         preferred_element_type=jnp.float32
                         ).astype(jnp.bfloat16)


def _l0_body(adj_ref, p_ref, b_ref, g_ref, beta_ref, w_ref,
             o_ref, adjb_ref):
    ab = adj_ref[...].astype(jnp.bfloat16)
    adjb_ref[...] = ab[:, _CF32:]
    acc = jnp.dot(ab, p_ref[...], preferred_element_type=jnp.float32)
    o_ref[...] = _ln_relu_proj(acc, b_ref, g_ref, beta_ref, w_ref)


def _mid_body(adjf_ref, adjb_ref, p_ref, b_ref, g_ref, beta_ref, w_ref,
              o_ref):
    nhi = adjb_ref.shape[1]
    acc = (jnp.dot(adjf_ref[...], p_ref[:_CF32, :],
                   preferred_element_type=jnp.float32)
           + jnp.dot(adjb_ref[...], p_ref[pl.ds(_CF32, nhi), :],
                     preferred_element_type=jnp.float32))
    o_ref[...] = _ln_relu_proj(acc, b_ref, g_ref, beta_ref, w_ref)


def _final_body(adjf_ref, adjb_ref, p_ref, b_ref, o_ref):
    nhi = adjb_ref.shape[1]
    acc = (jnp.dot(adjf_ref[...], p_ref[:_CF32, :],
                   preferred_element_type=jnp.float32)
           + jnp.dot(adjb_ref[...], p_ref[pl.ds(_CF32, nhi), :],
                     preferred_element_type=jnp.float32))
    h = acc + b_ref[...]
    m = jnp.max(h, axis=-1, keepdims=True)
    e = jnp.exp(h - m)
    lse = jnp.log(jnp.sum(e, axis=-1, keepdims=True)) + m
    o_ref[...] = h - lse


def kernel(x, adj_t, W0, b0, W1, b1, W2, b2, g1, beta1, g2, beta2):
    n, d = x.shape
    whi = n - _CF32
    r = lambda v: v.reshape(1, -1)
    vec = lambda: pl.BlockSpec((1, d), lambda i: (0, 0))

    p0 = pl.pallas_call(
        _proj_body,
        grid=(n // _BM_PROJ,),
        in_specs=[
            pl.BlockSpec((_BM_PROJ, d), lambda i: (i, 0)),
            pl.BlockSpec((d, d), lambda i: (0, 0)),
        ],
        out_specs=pl.BlockSpec((_BM_PROJ, d), lambda i: (i, 0)),
        out_shape=jax.ShapeDtypeStruct((n, d), jnp.bfloat16),
    )(x, W0)

    p1, adj_b = pl.pallas_call(
        _l0_body,
        grid=(n // _BM0,),
        in_specs=[
            pl.BlockSpec((_BM0, n), lambda i: (i, 0)),
            pl.BlockSpec((n, d), lambda i: (0, 0)),
            vec(), vec(), vec(),
            pl.BlockSpec((d, d), lambda i: (0, 0)),
        ],
        out_specs=[
            pl.BlockSpec((_BM0, d), lambda i: (i, 0)),
            pl.BlockSpec((_BM0, whi), lambda i: (i, 0)),
        ],
        out_shape=[
            jax.ShapeDtypeStruct((n, d), jnp.bfloat16),
            jax.ShapeDtypeStruct((n, whi), jnp.bfloat16),
        ],
    )(adj_t, p0, r(b0), r(g1), r(beta1), W1)

    mixed_specs = [
        pl.BlockSpec((_BM, _CF32), lambda i: (i, 0)),
        pl.BlockSpec((_BM, whi), lambda i: (i, 0)),
        pl.BlockSpec((n, d), lambda i: (0, 0)),
    ]

    p2 = pl.pallas_call(
        _mid_body,
        grid=(n // _BM,),
        in_specs=mixed_specs + [vec(), vec(), vec(),
                                pl.BlockSpec((d, d), lambda i: (0, 0))],
        out_specs=pl.BlockSpec((_BM, d), lambda i: (i, 0)),
        out_shape=jax.ShapeDtypeStruct((n, d), jnp.bfloat16),
    )(adj_t, adj_b, p1, r(b1), r(g2), r(beta2), W2)

    return pl.pallas_call(
        _final_body,
        grid=(n // _BM,),
        in_specs=mixed_specs + [vec()],
        out_specs=pl.BlockSpec((_BM, d), lambda i: (i, 0)),
        out_shape=jax.ShapeDtypeStruct((n, d), jnp.float32),
    )(adj_t, adj_b, p2, r(b2))
